# baseline (speedup 1.0000x reference)
import functools

import jax
import jax.numpy as jnp
from jax import lax
from jax.experimental import pallas as pl
from jax.experimental.pallas import tpu as pltpu
from jax.experimental.pallas import tpu_sc as plsc

_L = 16


def _make_sc_kernel(B, T, F, D, n_workers, b_per_w, nb):
    n_chunks = b_per_w // nb
    W = F + 2 * D
    nr = nb * T
    mesh = plsc.VectorSubcoreMesh(core_axis_name="c", subcore_axis_name="s")

    @functools.partial(
        pl.kernel,
        mesh=mesh,
        compiler_params=pltpu.CompilerParams(
            use_tc_tiling_on_sc=False, needs_layout_passes=False
        ),
        out_type=jax.ShapeDtypeStruct((B, T, W), jnp.float32),
        scratch_types=[
            pltpu.VMEM((nb, T, W), jnp.float32),
            pltpu.VMEM((nb, T, 2), jnp.int32),
            pltpu.VMEM((8, _L), jnp.float32),
            pltpu.VMEM((288, _L), jnp.float32),
            pltpu.SemaphoreType.DMA,
        ],
    )
    def k(inp_hbm, idx_hbm, wday_hbm, wtime_hbm, out_hbm,
          rows, idxp, wday_v, wtime_v, sem_inp):
        wid = lax.axis_index("s") * 2 + lax.axis_index("c")
        base = wid * b_per_w

        pltpu.sync_copy(wday_hbm, wday_v.at[pl.ds(0, 7), :])
        pltpu.sync_copy(wtime_hbm, wtime_v)

        iota = lax.iota(jnp.int32, _L)
        zero = jnp.zeros((_L,), jnp.int32)
        one = jnp.ones((_L,), jnp.int32)

        def chunk_body(ci, carry):
            b0 = base + ci * nb
            inp_cp = pltpu.make_async_copy(
                inp_hbm.at[pl.ds(b0, nb), :, :],
                rows.at[:, :, pl.ds(0, F)],
                sem_inp,
            )
            inp_cp.start()
            pltpu.sync_copy(idx_hbm.at[pl.ds(b0, nb), :, :], idxp)

            def group_body(g, c):
                r = iota + g * _L
                bv = lax.div(r, T)
                tv = r - bv * T
                d = plsc.load_gather(idxp, [bv, tv, zero])
                t = plsc.load_gather(idxp, [bv, tv, one])
                for col in range(D):
                    cvec = jnp.full((_L,), col, jnp.int32)
                    vd = plsc.load_gather(wday_v, [d, cvec])
                    plsc.store_scatter(
                        rows, [bv, tv, jnp.full((_L,), F + col, jnp.int32)],
                        vd)
                    vt = plsc.load_gather(wtime_v, [t, cvec])
                    plsc.store_scatter(
                        rows, [bv, tv, jnp.full((_L,), F + D + col, jnp.int32)],
                        vt)
                return c

            lax.fori_loop(0, nr // _L, group_body, 0)
            inp_cp.wait()
            pltpu.sync_copy(rows, out_hbm.at[pl.ds(b0, nb), :, :])
            return carry

        lax.fori_loop(0, n_chunks, chunk_body, 0)

    return k


def kernel(inp, daytime, W_day, W_time):
    B, T, F = inp.shape
    D = W_day.shape[1]
    n_workers = 32
    b_per_w = B // n_workers
    nb = 4
    assert b_per_w * n_workers == B and b_per_w % nb == 0
    assert (nb * T) % _L == 0

    idx2 = daytime.astype(jnp.int32)
    k = _make_sc_kernel(B, T, F, D, n_workers, b_per_w, nb)
    return k(inp, idx2, W_day, W_time)


# speedup vs baseline: 1.4815x; 1.4815x over previous
import functools

import jax
import jax.numpy as jnp
from jax import lax
from jax.experimental import pallas as pl
from jax.experimental.pallas import tpu as pltpu
from jax.experimental.pallas import tpu_sc as plsc

_L = 16


def _make_sc_kernel(B, T, F, D, n_workers, b_per_w):
    W = F + 2 * D
    n_full = T // _L
    tail = T - n_full * _L
    mesh = plsc.VectorSubcoreMesh(core_axis_name="c", subcore_axis_name="s")

    @functools.partial(
        pl.kernel,
        mesh=mesh,
        compiler_params=pltpu.CompilerParams(needs_layout_passes=False),
        out_type=jax.ShapeDtypeStruct((B, T, W), jnp.float32),
        scratch_types=[
            pltpu.VMEM((1, T, W), jnp.float32),
            pltpu.VMEM((1, T, F), jnp.float32),
            pltpu.VMEM((1, T, 2), jnp.int32),
            pltpu.VMEM((7, _L), jnp.float32),
            pltpu.VMEM((288, _L), jnp.float32),
            pltpu.SemaphoreType.DMA,
        ],
    )
    def k(inp_hbm, idx_hbm, wday_hbm, wtime_hbm, out_hbm,
          rows, ibuf, idxp, wday_v, wtime_v, sem_inp):
        wid = lax.axis_index("s") * 2 + lax.axis_index("c")
        base = wid * b_per_w

        pltpu.sync_copy(wday_hbm, wday_v)
        pltpu.sync_copy(wtime_hbm, wtime_v)

        iota = lax.iota(jnp.int32, _L)
        zero = jnp.zeros((_L,), jnp.int32)
        one = jnp.ones((_L,), jnp.int32)

        def emb_group(g, mask):
            tv = iota + g * _L
            tvc = jnp.minimum(tv, T - 1)
            d = plsc.load_gather(idxp, [zero, tvc, zero], mask=mask)
            t = plsc.load_gather(idxp, [zero, tvc, one], mask=mask)
            dc = jnp.clip(d, 0, 6)
            tc = jnp.clip(t, 0, 287)
            for col in range(D):
                cvec = jnp.full((_L,), col, jnp.int32)
                vd = plsc.load_gather(wday_v, [dc, cvec], mask=mask)
                plsc.store_scatter(
                    rows, [zero, tvc, jnp.full((_L,), F + col, jnp.int32)],
                    vd, mask=mask)
                vt = plsc.load_gather(wtime_v, [tc, cvec], mask=mask)
                plsc.store_scatter(
                    rows, [zero, tvc, jnp.full((_L,), F + D + col, jnp.int32)],
                    vt, mask=mask)

        full_mask = jnp.ones((_L,), jnp.bool_)
        tail_mask = iota < tail

        def chunk_body(b, carry):
            inp_cp = pltpu.make_async_copy(
                inp_hbm.at[pl.ds(base + b, 1), :, :], ibuf, sem_inp)
            inp_cp.start()
            pltpu.sync_copy(idx_hbm.at[pl.ds(base + b, 1), :, :], idxp)

            def emb_body(g, c):
                emb_group(g, full_mask)
                return c

            lax.fori_loop(0, n_full, emb_body, 0)
            if tail:
                emb_group(n_full, tail_mask)

            inp_cp.wait()

            def copy_body(t0, c):
                for dt in range(4):
                    t = t0 * 4 + dt
                    for c4 in range(F // _L):
                        rows[0, t, pl.ds(c4 * _L, _L)] = (
                            ibuf[0, t, pl.ds(c4 * _L, _L)])
                return c

            lax.fori_loop(0, T // 4, copy_body, 0)
            pltpu.sync_copy(rows, out_hbm.at[pl.ds(base + b, 1), :, :])
            return carry

        lax.fori_loop(0, b_per_w, chunk_body, 0)

    return k


def kernel(inp, daytime, W_day, W_time):
    B, T, F = inp.shape
    D = W_day.shape[1]
    n_workers = 32
    b_per_w = B // n_workers
    assert b_per_w * n_workers == B

    idx2 = daytime.astype(jnp.int32)
    k = _make_sc_kernel(B, T, F, D, n_workers, b_per_w)
    return k(inp, idx2, W_day, W_time)


# speedup vs baseline: 1.6976x; 1.1459x over previous
import functools

import jax
import jax.numpy as jnp
from jax import lax
from jax.experimental import pallas as pl
from jax.experimental.pallas import tpu as pltpu
from jax.experimental.pallas import tpu_sc as plsc

_L = 16


def _make_sc_kernel(B, T, F, D, n_workers, b_per_w):
    W = F + 2 * D
    n_full = T // _L
    tail = T - n_full * _L
    mesh = plsc.VectorSubcoreMesh(core_axis_name="c", subcore_axis_name="s")

    @functools.partial(
        pl.kernel,
        mesh=mesh,
        compiler_params=pltpu.CompilerParams(needs_layout_passes=False),
        out_type=jax.ShapeDtypeStruct((B, T, W), jnp.float32),
        scratch_types=[
            pltpu.VMEM((1, T, W), jnp.float32),
            pltpu.VMEM((1, T, W), jnp.float32),
            pltpu.VMEM((1, T, F), jnp.float32),
            pltpu.VMEM((1, T, 2), jnp.int32),
            pltpu.VMEM((7, _L), jnp.float32),
            pltpu.VMEM((36, 128), jnp.float32),
            pltpu.SemaphoreType.DMA,
            pltpu.SemaphoreType.DMA,
            pltpu.SemaphoreType.DMA,
            pltpu.SemaphoreType.DMA,
        ],
    )
    def k(inp_hbm, idx_hbm, wday_hbm, wtime_hbm, out_hbm,
          rows0, rows1, ibuf, idxp, wday_v, wtime_v,
          sem_inp, sem_idx, sem_o0, sem_o1):
        wid = lax.axis_index("s") * 2 + lax.axis_index("c")
        base = wid * b_per_w
        rows_bufs = (rows0, rows1)
        out_sems = (sem_o0, sem_o1)

        pltpu.sync_copy(wday_hbm, wday_v)
        pltpu.sync_copy(wtime_hbm, wtime_v)

        iota = lax.iota(jnp.int32, _L)
        zero = jnp.zeros((_L,), jnp.int32)
        one = jnp.ones((_L,), jnp.int32)

        def inp_cp(b, sl=None):
            return pltpu.make_async_copy(
                inp_hbm.at[pl.ds(b, 1), :, :], ibuf, sem_inp)

        def idx_cp(b):
            return pltpu.make_async_copy(
                idx_hbm.at[pl.ds(b, 1), :, :], idxp, sem_idx)

        def out_cp(p, b):
            return pltpu.make_async_copy(
                rows_bufs[p], out_hbm.at[pl.ds(b, 1), :, :], out_sems[p])

        def emb_group(rows, g, mask):
            tv = iota + g * _L
            tvc = jnp.minimum(tv, T - 1)
            d = plsc.load_gather(idxp, [zero, tvc, zero], mask=mask)
            t = plsc.load_gather(idxp, [zero, tvc, one], mask=mask)
            dc = jnp.clip(d, 0, 6)
            tc = jnp.clip(t, 0, 287)
            tq = lax.shift_right_logical(tc, 1 + 2)
            tbase = lax.shift_left(jnp.bitwise_and(tc, 7), 4)
            for col in range(D):
                cvec = jnp.full((_L,), col, jnp.int32)
                vd = plsc.load_gather(wday_v, [dc, cvec], mask=mask)
                plsc.store_scatter(
                    rows, [zero, tvc, jnp.full((_L,), F + col, jnp.int32)],
                    vd, mask=mask)
                vt = plsc.load_gather(wtime_v, [tq, tbase + cvec], mask=mask)
                plsc.store_scatter(
                    rows, [zero, tvc, jnp.full((_L,), F + D + col, jnp.int32)],
                    vt, mask=mask)

        full_mask = jnp.ones((_L,), jnp.bool_)
        tail_mask = iota < tail

        idx_cp(base).start()
        inp_cp(base).start()

        def super_body(j, carry):
            for p in range(2):
                b = base + j * 2 + p
                rows = rows_bufs[p]
                @pl.when(j >= 1)
                def _wait_prev():
                    out_cp(p, b).wait()

                idx_cp(b).wait()

                def emb_body(g, c):
                    emb_group(rows, g, full_mask)
                    return c

                lax.fori_loop(0, n_full, emb_body, 0)
                if tail:
                    emb_group(rows, n_full, tail_mask)

                @pl.when(b + 1 < base + b_per_w)
                def _pf_idx():
                    idx_cp(b + 1).start()

                inp_cp(b).wait()

                def copy_body(t0, c):
                    for dt in range(4):
                        t = t0 * 4 + dt
                        for c4 in range(F // _L):
                            rows[0, t, pl.ds(c4 * _L, _L)] = (
                                ibuf[0, t, pl.ds(c4 * _L, _L)])
                    return c

                lax.fori_loop(0, T // 4, copy_body, 0)

                @pl.when(b + 1 < base + b_per_w)
                def _pf_inp():
                    inp_cp(b + 1).start()

                out_cp(p, b).start()
            return carry

        lax.fori_loop(0, b_per_w // 2, super_body, 0)
        for p in range(2):
            out_cp(p, base + b_per_w - 2 + p).wait()

    return k


def kernel(inp, daytime, W_day, W_time):
    B, T, F = inp.shape
    D = W_day.shape[1]
    n_workers = 32
    b_per_w = B // n_workers
    assert b_per_w * n_workers == B and b_per_w % 2 == 0

    idx2 = daytime.astype(jnp.int32)
    wt128 = W_time.reshape(W_time.shape[0] * D // 128, 128)
    k = _make_sc_kernel(B, T, F, D, n_workers, b_per_w)
    return k(inp, idx2, W_day, wt128)


# speedup vs baseline: 4.9328x; 2.9058x over previous
import functools

import jax
import jax.numpy as jnp
from jax import lax
from jax.experimental import pallas as pl
from jax.experimental.pallas import tpu as pltpu
from jax.experimental.pallas import tpu_sc as plsc

_L = 16
_LANES = 128


def _make_sc_kernel(B, T, F, D, n_workers, tn):
    W = F + 2 * D
    n_chunks = T // tn
    mesh = plsc.VectorSubcoreMesh(core_axis_name="c", subcore_axis_name="s")

    @functools.partial(
        pl.kernel,
        mesh=mesh,
        compiler_params=pltpu.CompilerParams(needs_layout_passes=False),
        out_type=jax.ShapeDtypeStruct((T, W, B), jnp.float32),
        scratch_types=[
            pltpu.VMEM((tn, W, _LANES), jnp.float32),
            pltpu.VMEM((2 * tn, _LANES), jnp.int32),
            pltpu.VMEM((7, _L), jnp.float32),
            pltpu.VMEM((36, 128), jnp.float32),
            pltpu.SemaphoreType.DMA,
        ],
    )
    def k(inp_hbm, idx_hbm, wday_hbm, wtime_hbm, out_hbm,
          obuf, idxp, wday_v, wtime_v, sem_inp):
        wid = lax.axis_index("s") * 2 + lax.axis_index("c")
        lane0 = wid * _LANES

        pltpu.sync_copy(wday_hbm, wday_v)
        pltpu.sync_copy(wtime_hbm, wtime_v)

        def chunk_body(ci, carry):
            t0 = ci * tn
            inp_cp = pltpu.make_async_copy(
                inp_hbm.at[pl.ds(t0, tn), :, pl.ds(lane0, _LANES)],
                obuf.at[:, pl.ds(0, F), :],
                sem_inp,
            )
            inp_cp.start()
            pltpu.sync_copy(
                idx_hbm.at[pl.ds(2 * t0, 2 * tn), pl.ds(lane0, _LANES)],
                idxp)

            for tl in range(tn):
                for s in range(_LANES // _L):
                    sl = pl.ds(s * _L, _L)
                    d = idxp[2 * tl, sl]
                    t = idxp[2 * tl + 1, sl]
                    dc = jnp.clip(d, 0, 6)
                    tc = jnp.clip(t, 0, 287)
                    tq = lax.shift_right_logical(tc, 3)
                    tb = lax.shift_left(jnp.bitwise_and(tc, 7), 4)
                    for col in range(D):
                        cvec = jnp.full((_L,), col, jnp.int32)
                        obuf[tl, F + col, sl] = plsc.load_gather(
                            wday_v, [dc, cvec])
                        obuf[tl, F + D + col, sl] = plsc.load_gather(
                            wtime_v, [tq, tb + cvec])

            inp_cp.wait()
            pltpu.sync_copy(
                obuf, out_hbm.at[pl.ds(t0, tn), :, pl.ds(lane0, _LANES)])
            return carry

        lax.fori_loop(0, n_chunks, chunk_body, 0)

    return k


def kernel(inp, daytime, W_day, W_time):
    B, T, F = inp.shape
    D = W_day.shape[1]
    n_workers = 32
    assert B == n_workers * _LANES
    tn = 4
    assert T % tn == 0 and (2 * tn) % 8 == 0

    inp_t = jnp.transpose(inp, (1, 2, 0))
    idx_t = jnp.transpose(daytime.astype(jnp.int32), (1, 2, 0))
    idx2 = idx_t.reshape(2 * T, B)
    wt128 = W_time.reshape(W_time.shape[0] * D // 128, 128)
    k = _make_sc_kernel(B, T, F, D, n_workers, tn)
    out_t = k(inp_t, idx2, W_day, wt128)
    return jnp.transpose(out_t, (2, 0, 1))


# speedup vs baseline: 5.5752x; 1.1302x over previous
import functools

import jax
import jax.numpy as jnp
from jax import lax
from jax.experimental import pallas as pl
from jax.experimental.pallas import tpu as pltpu
from jax.experimental.pallas import tpu_sc as plsc

_L = 16
_LANES = 128


def _make_sc_kernel(B, T, F, D, n_workers, tn):
    W = F + 2 * D
    n_chunks = T // tn
    mesh = plsc.VectorSubcoreMesh(core_axis_name="c", subcore_axis_name="s")

    @functools.partial(
        pl.kernel,
        mesh=mesh,
        compiler_params=pltpu.CompilerParams(needs_layout_passes=False),
        out_type=jax.ShapeDtypeStruct((T, W, B), jnp.float32),
        scratch_types=[
            pltpu.VMEM((tn, W, _LANES), jnp.float32),
            pltpu.VMEM((tn, W, _LANES), jnp.float32),
            pltpu.VMEM((2 * tn, _LANES), jnp.int32),
            pltpu.VMEM((7, _L), jnp.float32),
            pltpu.VMEM((36, 128), jnp.float32),
            pltpu.SemaphoreType.DMA,
            pltpu.SemaphoreType.DMA,
            pltpu.SemaphoreType.DMA,
        ],
    )
    def k(inp_hbm, idx_hbm, wday_hbm, wtime_hbm, out_hbm,
          obuf0, obuf1, idxp, wday_v, wtime_v, sem_inp, sem_o0, sem_o1):
        wid = lax.axis_index("s") * 2 + lax.axis_index("c")
        lane0 = wid * _LANES
        obufs = (obuf0, obuf1)
        out_sems = (sem_o0, sem_o1)

        pltpu.sync_copy(wday_hbm, wday_v)
        pltpu.sync_copy(wtime_hbm, wtime_v)

        def out_cp(p, ci):
            t0 = ci * tn
            return pltpu.make_async_copy(
                obufs[p],
                out_hbm.at[pl.ds(t0, tn), :, pl.ds(lane0, _LANES)],
                out_sems[p])

        def do_chunk(j, p):
            ci = j * 2 + p
            t0 = ci * tn
            obuf = obufs[p]

            @pl.when(j >= 1)
            def _wait_prev():
                out_cp(p, ci - 2).wait()

            inp_cp = pltpu.make_async_copy(
                inp_hbm.at[pl.ds(t0, tn), :, pl.ds(lane0, _LANES)],
                obuf.at[:, pl.ds(0, F), :],
                sem_inp,
            )
            inp_cp.start()
            pltpu.sync_copy(
                idx_hbm.at[pl.ds(2 * t0, 2 * tn), pl.ds(lane0, _LANES)],
                idxp)

            for tl in range(tn):
                for s in range(_LANES // _L):
                    sl = pl.ds(s * _L, _L)
                    d = idxp[2 * tl, sl]
                    t = idxp[2 * tl + 1, sl]
                    dc = jnp.clip(d, 0, 6)
                    tc = jnp.clip(t, 0, 287)
                    tq = lax.shift_right_logical(tc, 3)
                    tb = lax.shift_left(jnp.bitwise_and(tc, 7), 4)
                    for col in range(D):
                        cvec = jnp.full((_L,), col, jnp.int32)
                        obuf[tl, F + col, sl] = plsc.load_gather(
                            wday_v, [dc, cvec])
                        obuf[tl, F + D + col, sl] = plsc.load_gather(
                            wtime_v, [tq, tb + cvec])

            inp_cp.wait()
            out_cp(p, ci).start()

        def super_body(j, carry):
            for p in range(2):
                do_chunk(j, p)
            return carry

        lax.fori_loop(0, n_chunks // 2, super_body, 0)
        for p in range(2):
            out_cp(p, n_chunks - 2 + p).wait()

    return k


def kernel(inp, daytime, W_day, W_time):
    B, T, F = inp.shape
    D = W_day.shape[1]
    n_workers = 32
    assert B == n_workers * _LANES
    tn = 4
    assert T % tn == 0 and (T // tn) % 2 == 0 and (2 * tn) % 8 == 0

    inp_t = jnp.transpose(inp, (1, 2, 0))
    idx_t = jnp.transpose(daytime.astype(jnp.int32), (1, 2, 0))
    idx2 = idx_t.reshape(2 * T, B)
    wt128 = W_time.reshape(W_time.shape[0] * D // 128, 128)
    k = _make_sc_kernel(B, T, F, D, n_workers, tn)
    out_t = k(inp_t, idx2, W_day, wt128)
    return jnp.transpose(out_t, (2, 0, 1))


# speedup vs baseline: 5.9808x; 1.0728x over previous
import functools

import jax
import jax.numpy as jnp
from jax import lax
from jax.experimental import pallas as pl
from jax.experimental.pallas import tpu as pltpu
from jax.experimental.pallas import tpu_sc as plsc

_L = 16
_LANES = 128


def _make_sc_kernel(B, T, F, D, n_workers, tn):
    W = F + 2 * D
    n_chunks = T // tn
    mesh = plsc.VectorSubcoreMesh(core_axis_name="c", subcore_axis_name="s")

    @functools.partial(
        pl.kernel,
        mesh=mesh,
        compiler_params=pltpu.CompilerParams(needs_layout_passes=False),
        out_type=jax.ShapeDtypeStruct((T, W, B), jnp.float32),
        scratch_types=[
            pltpu.VMEM((tn, W, _LANES), jnp.float32),
            pltpu.VMEM((tn, W, _LANES), jnp.float32),
            pltpu.VMEM((2 * tn, _LANES), jnp.int32),
            pltpu.VMEM((2 * tn, _LANES), jnp.int32),
            pltpu.VMEM((7, _L), jnp.float32),
            pltpu.VMEM((36, 128), jnp.float32),
            pltpu.SemaphoreType.DMA,
            pltpu.SemaphoreType.DMA,
            pltpu.SemaphoreType.DMA,
            pltpu.SemaphoreType.DMA,
        ],
    )
    def k(inp_hbm, idx_hbm, wday_hbm, wtime_hbm, out_hbm,
          obuf0, obuf1, idxp0, idxp1, wday_v, wtime_v,
          sem_inp, sem_idx, sem_o0, sem_o1):
        wid = lax.axis_index("s") * 2 + lax.axis_index("c")
        lane0 = wid * _LANES
        obufs = (obuf0, obuf1)
        idxps = (idxp0, idxp1)
        out_sems = (sem_o0, sem_o1)

        pltpu.sync_copy(wday_hbm, wday_v)
        pltpu.sync_copy(wtime_hbm, wtime_v)

        def out_cp(p, ci):
            t0 = ci * tn
            return pltpu.make_async_copy(
                obufs[p],
                out_hbm.at[pl.ds(t0, tn), :, pl.ds(lane0, _LANES)],
                out_sems[p])

        def idx_cp(p, ci):
            t0 = ci * tn
            return pltpu.make_async_copy(
                idx_hbm.at[pl.ds(2 * t0, 2 * tn), pl.ds(lane0, _LANES)],
                idxps[p], sem_idx)

        def do_chunk(j, p):
            ci = j * 2 + p
            t0 = ci * tn
            obuf = obufs[p]
            idxp = idxps[p]

            @pl.when(j >= 1)
            def _wait_prev():
                out_cp(p, ci - 2).wait()

            inp_cp = pltpu.make_async_copy(
                inp_hbm.at[pl.ds(t0, tn), :, pl.ds(lane0, _LANES)],
                obuf.at[:, pl.ds(0, F), :],
                sem_inp,
            )
            inp_cp.start()
            idx_cp(p, ci).wait()

            @pl.when(ci + 1 < n_chunks)
            def _pf_idx():
                idx_cp(1 - p, ci + 1).start()

            for tl in range(tn):
                for s in range(_LANES // _L):
                    sl = pl.ds(s * _L, _L)
                    d = idxp[2 * tl, sl]
                    t = idxp[2 * tl + 1, sl]
                    dc = jnp.clip(d, 0, 6)
                    tc = jnp.clip(t, 0, 287)
                    tq = lax.shift_right_logical(tc, 3)
                    tb = lax.shift_left(jnp.bitwise_and(tc, 7), 4)
                    for col in range(D):
                        cvec = jnp.full((_L,), col, jnp.int32)
                        obuf[tl, F + col, sl] = plsc.load_gather(
                            wday_v, [dc, cvec])
                        obuf[tl, F + D + col, sl] = plsc.load_gather(
                            wtime_v, [tq, tb + cvec])

            inp_cp.wait()
            out_cp(p, ci).start()

        def super_body(j, carry):
            for p in range(2):
                do_chunk(j, p)
            return carry

        idx_cp(0, 0).start()
        lax.fori_loop(0, n_chunks // 2, super_body, 0)
        for p in range(2):
            out_cp(p, n_chunks - 2 + p).wait()

    return k


def kernel(inp, daytime, W_day, W_time):
    B, T, F = inp.shape
    D = W_day.shape[1]
    n_workers = 32
    assert B == n_workers * _LANES
    tn = 4
    assert T % tn == 0 and (T // tn) % 2 == 0 and (2 * tn) % 8 == 0

    inp_t = jnp.transpose(inp, (1, 2, 0))
    idx_t = jnp.transpose(daytime.astype(jnp.int32), (1, 2, 0))
    idx2 = idx_t.reshape(2 * T, B)
    wt128 = W_time.reshape(W_time.shape[0] * D // 128, 128)
    k = _make_sc_kernel(B, T, F, D, n_workers, tn)
    out_t = k(inp_t, idx2, W_day, wt128)
    return jnp.transpose(out_t, (2, 0, 1))
